# baseline (speedup 1.0000x reference)
import functools

import jax
import jax.numpy as jnp
from jax import lax
from jax.experimental import pallas as pl
from jax.experimental.pallas import tpu as pltpu
from jax.experimental.pallas import tpu_sc as plsc

_VOCAB = 1000000
_DIM = 64
_BATCH = 16384

_NC = 2
_NS = 16
_NW = _NC * _NS

_CHUNK = 128
_NCHUNKS = 3 * _BATCH // _CHUNK
_CPW = _NCHUNKS // _NW
_NBUF = 3

_mesh = plsc.VectorSubcoreMesh(core_axis_name="c", subcore_axis_name="s")


@functools.partial(
    pl.kernel,
    out_type=jax.ShapeDtypeStruct((_NW, _CPW, _CHUNK, 2 * _DIM), jnp.float32),
    mesh=_mesh,
    compiler_params=pltpu.CompilerParams(use_tc_tiling_on_sc=True),
    scratch_types=[
        pltpu.VMEM((_CPW, _CHUNK), jnp.int32),
        pltpu.VMEM((_NBUF, _CHUNK, 2 * _DIM), jnp.float32),
        pltpu.SemaphoreType.DMA,
    ],
)
def _gather_kernel(table_hbm, idx_hbm, out_hbm, idx_v, bufs_v, sem):
    wid = lax.axis_index("s") * _NC + lax.axis_index("c")
    pltpu.sync_copy(idx_hbm.at[wid], idx_v)
    gathers = [None] * _CPW
    for j in range(_NBUF):
        gathers[j] = pltpu.async_copy(
            table_hbm.at[idx_v.at[j]], bufs_v.at[j % _NBUF], sem
        )
    for j in range(_CPW):
        gathers[j].wait()
        pltpu.sync_copy(bufs_v.at[j % _NBUF], out_hbm.at[wid, j])
        nxt = j + _NBUF
        if nxt < _CPW:
            gathers[nxt] = pltpu.async_copy(
                table_hbm.at[idx_v.at[nxt]], bufs_v.at[nxt % _NBUF], sem
            )


def kernel(anchor, pos, neg, embedding_table):
    table2 = embedding_table.reshape(_VOCAB // 2, 2 * _DIM)
    idx = jnp.concatenate([anchor, pos, neg]).astype(jnp.int32)
    order = jnp.argsort(idx)
    sorted_ids = idx[order]
    phys = (sorted_ids // 2).reshape(_NW, _CPW, _CHUNK)
    wide = _gather_kernel(table2, phys)
    return (wide, wide, order)


# speedup vs baseline: 2.0108x; 2.0108x over previous
import functools

import jax
import jax.numpy as jnp
from jax import lax
from jax.experimental import pallas as pl
from jax.experimental.pallas import tpu as pltpu
from jax.experimental.pallas import tpu_sc as plsc

_VOCAB = 1000000
_DIM = 64
_BATCH = 16384
_TOTAL = 3 * _BATCH

_NC = 2
_NS = 16
_NW = _NC * _NS
_HPW = _TOTAL // _NW

_LBLK = 512
_NFULL = _VOCAB // _LBLK
_TAIL = _VOCAB - _NFULL * _LBLK

_RING = 8

_mesh = plsc.VectorSubcoreMesh(core_axis_name="c", subcore_axis_name="s")


@functools.partial(
    pl.kernel,
    out_type=jax.ShapeDtypeStruct((_TOTAL * _DIM,), jnp.float32),
    mesh=_mesh,
    compiler_params=pltpu.CompilerParams(
        use_tc_tiling_on_sc=True, needs_layout_passes=False
    ),
    scratch_types=[
        pltpu.VMEM((_HPW // 128, 128), jnp.int32),
        pltpu.VMEM((_HPW // 128, 128), jnp.int32),
        pltpu.VMEM((_DIM, _LBLK), jnp.float32),
        pltpu.VMEM((_RING, _DIM), jnp.float32),
        pltpu.SemaphoreType.DMA,
    ],
)
def _gather_kernel(tableT, tail_pad, ids_hbm, pos_hbm, out_hbm, ids_v, pos_v,
                   blk_v, ring_v, sem_out):
    wid = lax.axis_index("s") * _NC + lax.axis_index("c")
    pltpu.sync_copy(ids_hbm.at[wid], ids_v)
    pltpu.sync_copy(pos_hbm.at[wid], pos_v)

    first_id = ids_v[0, pl.ds(0, 16)][0]
    first_pos = pos_v[0, pl.ds(0, 16)][0]
    last_id = ids_v[_HPW // 128 - 1, pl.ds(112, 16)][15]
    b_lo = first_id // _LBLK
    b_hi = last_id // _LBLK

    lane = lax.iota(jnp.int32, 16)

    def read_elem(ref, h):
        r = jnp.full((16,), h // 128, jnp.int32)
        c = jnp.full((16,), lax.rem(h, jnp.int32(128)), jnp.int32)
        return plsc.load_gather(ref, [r, c])[0]

    def drain_one_row():
        pltpu.make_async_copy(
            out_hbm.at[pl.ds(0, _DIM)], ring_v.at[0], sem_out
        ).wait()

    def process_hit(h, carry):
        cur_blk, nid, npos = carry
        blk = nid // _LBLK

        @pl.when(jnp.logical_and(blk != cur_blk, blk != _NFULL))
        def _():
            pltpu.sync_copy(tableT.at[:, pl.ds(blk * _LBLK, _LBLK)], blk_v)

        @pl.when(jnp.logical_and(blk != cur_blk, blk == _NFULL))
        def _():
            pltpu.sync_copy(tail_pad, blk_v)

        c = nid - blk * _LBLK
        slot = lax.rem(h, jnp.int32(_RING))

        drain_one_row()

        col = jnp.full((16,), c, jnp.int32)
        for g in range(_DIM // 16):
            v = plsc.load_gather(blk_v, [lane + (16 * g), col])
            ring_v[slot, pl.ds(16 * g, 16)] = v
        pltpu.async_copy(
            ring_v.at[slot], out_hbm.at[pl.ds(npos * _DIM, _DIM)], sem_out
        )

        h1 = h + 1
        hr = jnp.minimum(h1, _HPW - 1)
        nid1 = read_elem(ids_v, hr)
        npos1 = read_elem(pos_v, hr)
        return (blk, nid1, npos1)

    for s in range(_RING):
        pltpu.async_copy(out_hbm.at[pl.ds(0, _DIM)], ring_v.at[s], sem_out)

    carry0 = (jnp.int32(-1), first_id, first_pos)
    lax.fori_loop(0, _HPW, process_hit, carry0)

    for _ in range(_RING):
        drain_one_row()


def kernel(anchor, pos, neg, embedding_table):
    tableT = embedding_table.T
    tail_pad = jnp.zeros((_DIM, _LBLK), jnp.float32)
    tail_pad = tail_pad.at[:, :_TAIL].set(embedding_table[_NFULL * _LBLK:, :].T)
    idx = jnp.concatenate([anchor, pos, neg]).astype(jnp.int32)
    order = jnp.argsort(idx).astype(jnp.int32)
    sorted_ids = idx[order]
    out = _gather_kernel(
        tableT,
        tail_pad,
        sorted_ids.reshape(_NW, _HPW // 128, 128),
        order.reshape(_NW, _HPW // 128, 128),
    )
    out = out.reshape(3, _BATCH * _DIM, 1)
    return out[0], out[1], out[2]


# speedup vs baseline: 2.7143x; 1.3499x over previous
import functools

import jax
import jax.numpy as jnp
from jax import lax
from jax.experimental import pallas as pl
from jax.experimental.pallas import tpu as pltpu
from jax.experimental.pallas import tpu_sc as plsc

_VOCAB = 1000000
_DIM = 64
_BATCH = 16384
_TOTAL = 3 * _BATCH

_NC = 2
_NS = 16
_NW = _NC * _NS
_HPW = _TOTAL // _NW

_LBLK = 512
_NFULL = _VOCAB // _LBLK
_TAIL = _VOCAB - _NFULL * _LBLK

_RING = 16

_mesh = plsc.VectorSubcoreMesh(core_axis_name="c", subcore_axis_name="s")


@functools.partial(
    pl.kernel,
    out_type=jax.ShapeDtypeStruct((_TOTAL * _DIM,), jnp.float32),
    mesh=_mesh,
    compiler_params=pltpu.CompilerParams(
        use_tc_tiling_on_sc=True, needs_layout_passes=False
    ),
    scratch_types=[
        pltpu.VMEM((_HPW // 128, 128), jnp.int32),
        pltpu.VMEM((_HPW // 128, 128), jnp.int32),
        pltpu.VMEM((2, _DIM, _LBLK), jnp.float32),
        pltpu.VMEM((_RING, _DIM), jnp.float32),
        pltpu.SemaphoreType.DMA,
        pltpu.SemaphoreType.DMA,
    ],
)
def _gather_kernel(tableT, tail_pad, ids_hbm, pos_hbm, out_hbm, ids_v, pos_v,
                   bufs_v, ring_v, sem_out, sem_blk):
    wid = lax.axis_index("s") * _NC + lax.axis_index("c")
    pltpu.sync_copy(ids_hbm.at[wid], ids_v)
    pltpu.sync_copy(pos_hbm.at[wid], pos_v)

    first_id = ids_v[0, pl.ds(0, 16)][0]
    lane = lax.iota(jnp.int32, 16)

    def drain_one_row():
        pltpu.make_async_copy(
            out_hbm.at[pl.ds(0, _DIM)], ring_v.at[0], sem_out
        ).wait()

    def wait_block():
        pltpu.make_async_copy(
            tableT.at[:, pl.ds(0, _LBLK)], bufs_v.at[0], sem_blk
        ).wait()

    def prefetch_block(pred, b, buf_slot):
        @pl.when(jnp.logical_and(pred, b != _NFULL))
        def _():
            pltpu.async_copy(
                tableT.at[:, pl.ds(b * _LBLK, _LBLK)],
                bufs_v.at[buf_slot], sem_blk,
            )

        @pl.when(jnp.logical_and(pred, b == _NFULL))
        def _():
            pltpu.async_copy(tail_pad, bufs_v.at[buf_slot], sem_blk)

    def process_group(g, carry):
        cur_blk, parity, pid = carry
        row = jnp.full((16,), g // 8, jnp.int32)
        colg = (lax.rem(g, jnp.int32(8)) * 16) + lane
        ids16 = plsc.load_gather(ids_v, [row, colg])
        pos16 = plsc.load_gather(pos_v, [row, colg])

        for j in range(16):
            nid = ids16[j]
            npos = pos16[j]
            blk = nid // _LBLK
            switch = blk != cur_blk
            hit_pf = jnp.logical_and(switch, pid == blk)
            gap = jnp.logical_and(switch, pid != blk)

            @pl.when(switch)
            def _():
                wait_block()

            @pl.when(jnp.logical_and(gap, blk != _NFULL))
            def _():
                pltpu.sync_copy(
                    tableT.at[:, pl.ds(blk * _LBLK, _LBLK)],
                    bufs_v.at[parity],
                )

            @pl.when(jnp.logical_and(gap, blk == _NFULL))
            def _():
                pltpu.sync_copy(tail_pad, bufs_v.at[parity])

            parity = jnp.where(hit_pf, 1 - parity, parity)
            cur_blk = jnp.where(switch, blk, cur_blk)
            tp = jnp.minimum(blk + 1, _NFULL)
            prefetch_block(switch, tp, 1 - parity)
            pid = jnp.where(switch, tp, pid)

            c = nid - blk * _LBLK
            slot = j

            drain_one_row()

            par16 = jnp.full((16,), parity, jnp.int32)
            col = jnp.full((16,), c, jnp.int32)
            for q in range(_DIM // 16):
                v = plsc.load_gather(bufs_v, [par16, lane + (16 * q), col])
                ring_v[slot, pl.ds(16 * q, 16)] = v
            pltpu.async_copy(
                ring_v.at[slot], out_hbm.at[pl.ds(npos * _DIM, _DIM)],
                sem_out,
            )

        return (cur_blk, parity, pid)

    for s in range(_RING):
        pltpu.async_copy(out_hbm.at[pl.ds(0, _DIM)], ring_v.at[s], sem_out)

    b0 = first_id // _LBLK

    @pl.when(b0 != _NFULL)
    def _():
        pltpu.sync_copy(tableT.at[:, pl.ds(b0 * _LBLK, _LBLK)], bufs_v.at[0])

    @pl.when(b0 == _NFULL)
    def _():
        pltpu.sync_copy(tail_pad, bufs_v.at[0])

    tp0 = jnp.minimum(b0 + 1, _NFULL)
    prefetch_block(jnp.bool_(True), tp0, 1)

    carry0 = (b0, jnp.int32(0), tp0)
    lax.fori_loop(0, _HPW // 16, process_group, carry0)

    wait_block()
    for _ in range(_RING):
        drain_one_row()


def kernel(anchor, pos, neg, embedding_table):
    tableT = embedding_table.T
    tail_pad = jnp.zeros((_DIM, _LBLK), jnp.float32)
    tail_pad = tail_pad.at[:, :_TAIL].set(embedding_table[_NFULL * _LBLK:, :].T)
    idx = jnp.concatenate([anchor, pos, neg]).astype(jnp.int32)
    order = jnp.argsort(idx).astype(jnp.int32)
    sorted_ids = idx[order]
    out = _gather_kernel(
        tableT,
        tail_pad,
        sorted_ids.reshape(_NW, _HPW // 128, 128),
        order.reshape(_NW, _HPW // 128, 128),
    )
    out = out.reshape(3, _BATCH * _DIM, 1)
    return out[0], out[1], out[2]


# speedup vs baseline: 2.8589x; 1.0533x over previous
import functools

import jax
import jax.numpy as jnp
from jax import lax
from jax.experimental import pallas as pl
from jax.experimental.pallas import tpu as pltpu
from jax.experimental.pallas import tpu_sc as plsc

_VOCAB = 1000000
_DIM = 64
_BATCH = 16384
_TOTAL = 3 * _BATCH

_NC = 2
_NS = 16
_NW = _NC * _NS
_HPW = _TOTAL // _NW

_LBLK = 768
_NFULL = _VOCAB // _LBLK
_TAIL = _VOCAB - _NFULL * _LBLK

_RING = 16

_mesh = plsc.VectorSubcoreMesh(core_axis_name="c", subcore_axis_name="s")


@functools.partial(
    pl.kernel,
    out_type=jax.ShapeDtypeStruct((_TOTAL * _DIM,), jnp.float32),
    mesh=_mesh,
    compiler_params=pltpu.CompilerParams(
        use_tc_tiling_on_sc=True, needs_layout_passes=False
    ),
    scratch_types=[
        pltpu.VMEM((_HPW // 128, 128), jnp.int32),
        pltpu.VMEM((_HPW // 128, 128), jnp.int32),
        pltpu.VMEM((2, _DIM, _LBLK), jnp.float32),
        pltpu.VMEM((_RING, _DIM), jnp.float32),
        pltpu.SemaphoreType.DMA,
        pltpu.SemaphoreType.DMA,
    ],
)
def _gather_kernel(tableT, tail_pad, ids_hbm, pos_hbm, out_hbm, ids_v, pos_v,
                   bufs_v, ring_v, sem_out, sem_blk):
    wid = lax.axis_index("s") * _NC + lax.axis_index("c")
    pltpu.sync_copy(ids_hbm.at[wid], ids_v)
    pltpu.sync_copy(pos_hbm.at[wid], pos_v)

    first_id = ids_v[0, pl.ds(0, 16)][0]
    lane = lax.iota(jnp.int32, 16)

    def drain_one_row():
        pltpu.make_async_copy(
            out_hbm.at[pl.ds(0, _DIM)], ring_v.at[0], sem_out
        ).wait()

    def wait_block():
        pltpu.make_async_copy(
            tableT.at[:, pl.ds(0, _LBLK)], bufs_v.at[0], sem_blk
        ).wait()

    def prefetch_block(pred, b, buf_slot):
        @pl.when(jnp.logical_and(pred, b != _NFULL))
        def _():
            pltpu.async_copy(
                tableT.at[:, pl.ds(b * _LBLK, _LBLK)],
                bufs_v.at[buf_slot], sem_blk,
            )

        @pl.when(jnp.logical_and(pred, b == _NFULL))
        def _():
            pltpu.async_copy(tail_pad, bufs_v.at[buf_slot], sem_blk)

    def process_group(g, carry):
        cur_blk, parity, pid = carry
        row = jnp.full((16,), g // 8, jnp.int32)
        colg = (lax.rem(g, jnp.int32(8)) * 16) + lane
        ids16 = plsc.load_gather(ids_v, [row, colg])
        pos16 = plsc.load_gather(pos_v, [row, colg])

        for j in range(16):
            nid = ids16[j]
            npos = pos16[j]
            blk = nid // _LBLK
            switch = blk != cur_blk
            hit_pf = jnp.logical_and(switch, pid == blk)
            gap = jnp.logical_and(switch, pid != blk)

            @pl.when(switch)
            def _():
                wait_block()

            @pl.when(jnp.logical_and(gap, blk != _NFULL))
            def _():
                pltpu.sync_copy(
                    tableT.at[:, pl.ds(blk * _LBLK, _LBLK)],
                    bufs_v.at[parity],
                )

            @pl.when(jnp.logical_and(gap, blk == _NFULL))
            def _():
                pltpu.sync_copy(tail_pad, bufs_v.at[parity])

            parity = jnp.where(hit_pf, 1 - parity, parity)
            cur_blk = jnp.where(switch, blk, cur_blk)
            tp = jnp.minimum(blk + 1, _NFULL)
            prefetch_block(switch, tp, 1 - parity)
            pid = jnp.where(switch, tp, pid)

            c = nid - blk * _LBLK
            slot = j

            drain_one_row()

            par16 = jnp.full((16,), parity, jnp.int32)
            col = jnp.full((16,), c, jnp.int32)
            for q in range(_DIM // 16):
                v = plsc.load_gather(bufs_v, [par16, lane + (16 * q), col])
                ring_v[slot, pl.ds(16 * q, 16)] = v
            pltpu.async_copy(
                ring_v.at[slot], out_hbm.at[pl.ds(npos * _DIM, _DIM)],
                sem_out,
            )

        return (cur_blk, parity, pid)

    for s in range(_RING):
        pltpu.async_copy(out_hbm.at[pl.ds(0, _DIM)], ring_v.at[s], sem_out)

    b0 = first_id // _LBLK

    @pl.when(b0 != _NFULL)
    def _():
        pltpu.sync_copy(tableT.at[:, pl.ds(b0 * _LBLK, _LBLK)], bufs_v.at[0])

    @pl.when(b0 == _NFULL)
    def _():
        pltpu.sync_copy(tail_pad, bufs_v.at[0])

    tp0 = jnp.minimum(b0 + 1, _NFULL)
    prefetch_block(jnp.bool_(True), tp0, 1)

    carry0 = (b0, jnp.int32(0), tp0)
    lax.fori_loop(0, _HPW // 16, process_group, carry0)

    wait_block()
    for _ in range(_RING):
        drain_one_row()


def kernel(anchor, pos, neg, embedding_table):
    tableT = embedding_table.T
    tail_pad = jnp.zeros((_DIM, _LBLK), jnp.float32)
    tail_pad = tail_pad.at[:, :_TAIL].set(embedding_table[_NFULL * _LBLK:, :].T)
    idx = jnp.concatenate([anchor, pos, neg]).astype(jnp.int32)
    sorted_ids, order = lax.sort_key_val(
        idx, lax.iota(jnp.int32, _TOTAL)
    )
    out = _gather_kernel(
        tableT,
        tail_pad,
        sorted_ids.reshape(_NW, _HPW // 128, 128),
        order.reshape(_NW, _HPW // 128, 128),
    )
    out = out.reshape(3, _BATCH * _DIM, 1)
    return out[0], out[1], out[2]
